# Initial kernel scaffold; baseline (speedup 1.0000x reference)
import jax
import jax.numpy as jnp
from jax.experimental import pallas as pl


def kernel(x_all, edge_index, p, fc1_W, fc1_b, ln1_g, ln1_b, hW0, hb0, eW0, eb0, eg0, ebt0, hW1, hb1, eW1, eb1, eg1, ebt1, hW2, hb2, eW2, eb2, eg2, ebt2, fc3_W, fc3_b, ln3_g, ln3_b, fc_W, fc_b):
    raise NotImplementedError("write your pallas kernel here")


# speedup vs baseline: 6.0642x; 6.0642x over previous
import functools

import jax
import jax.numpy as jnp
from jax import lax
from jax.experimental import pallas as pl
from jax.experimental.pallas import tpu as pltpu
from jax.experimental.pallas import tpu_sc as plsc

HID = 256
N = 10000
NP = 10240
E = 160000
EP = 163840
LAYERS = 3
NUM_CLASS = 2

NS = 16
NC = 2
CHUNK = 128
EPT = EP // NS
NCHUNK = EPT // CHUNK
RPT = NP // NS
RCH = RPT // CHUNK

_F32 = jnp.float32


def _make_seg_sum(with_deg: bool):
    mesh = plsc.VectorSubcoreMesh(core_axis_name="c", subcore_axis_name="s")

    out_type = [
        jax.ShapeDtypeStruct((NP, 128), _F32),
        jax.ShapeDtypeStruct((NP, 128), _F32),
    ]
    scratch = [
        pltpu.VMEM((CHUNK,), jnp.int32),
        pltpu.VMEM((CHUNK,), jnp.int32),
        pltpu.VMEM((CHUNK, 128), _F32),
        pltpu.VMEM_SHARED((NP, 128), _F32),
        pltpu.SemaphoreType.DMA,
    ]
    if with_deg:
        out_type.append(jax.ShapeDtypeStruct((NP, 16), _F32))
        scratch += [
            pltpu.VMEM((CHUNK, 16), _F32),
            pltpu.VMEM((CHUNK, 16), _F32),
            pltpu.VMEM_SHARED((NP, 16), _F32),
        ]

    def body(y_lo, y_hi, src_hbm, dst_hbm, *refs):
        if with_deg:
            (out_lo, out_hi, deg_out,
             src_v, dst_v, rows_v, acc, sem, ones_v, dbuf_v, deg_acc) = refs
        else:
            out_lo, out_hi, src_v, dst_v, rows_v, acc, sem = refs

        c = lax.axis_index("c")
        s = lax.axis_index("s")

        zeros16 = jnp.zeros((16,), _F32)

        def _zero_row(r, _):
            def _zero_col(cc, _):
                rows_v[r, pl.ds(cc * 16, 16)] = zeros16
                return 0
            return lax.fori_loop(0, 128 // 16, _zero_col, 0)

        lax.fori_loop(0, CHUNK, _zero_row, 0)

        if with_deg:
            ones16 = jnp.ones((16,), _F32)

            def _fill_deg(r, _):
                ones_v[r, :] = ones16
                dbuf_v[r, :] = zeros16
                return 0

            lax.fori_loop(0, CHUNK, _fill_deg, 0)

        base = s * RPT
        for j in range(RCH):
            pltpu.sync_copy(rows_v, acc.at[pl.ds(base + j * CHUNK, CHUNK)])
        if with_deg:
            for j in range(RCH):
                pltpu.sync_copy(dbuf_v, deg_acc.at[pl.ds(base + j * CHUNK, CHUNK)])
        plsc.subcore_barrier()

        def edge_body(k, _):
            off = s * EPT + k * CHUNK
            pltpu.sync_copy(src_hbm.at[pl.ds(off, CHUNK)], src_v)
            pltpu.sync_copy(dst_hbm.at[pl.ds(off, CHUNK)], dst_v)

            @pl.when(c == 0)
            def _():
                pltpu.async_copy(y_lo.at[src_v], rows_v, sem).wait()

            @pl.when(c == 1)
            def _():
                pltpu.async_copy(y_hi.at[src_v], rows_v, sem).wait()

            pltpu.sync_copy(rows_v, acc.at[dst_v], add=True)
            if with_deg:
                @pl.when(c == 0)
                def _():
                    pltpu.sync_copy(ones_v, deg_acc.at[dst_v], add=True)
            return 0

        lax.fori_loop(0, NCHUNK, edge_body, 0)
        plsc.subcore_barrier()

        for j in range(RCH):
            r0 = base + j * CHUNK
            pltpu.sync_copy(acc.at[pl.ds(r0, CHUNK)], rows_v)

            @pl.when(c == 0)
            def _():
                pltpu.sync_copy(rows_v, out_lo.at[pl.ds(r0, CHUNK)])

            @pl.when(c == 1)
            def _():
                pltpu.sync_copy(rows_v, out_hi.at[pl.ds(r0, CHUNK)])

        if with_deg:
            @pl.when(c == 0)
            def _():
                for j in range(RCH):
                    r0 = base + j * CHUNK
                    pltpu.sync_copy(deg_acc.at[pl.ds(r0, CHUNK)], dbuf_v)
                    pltpu.sync_copy(dbuf_v, deg_out.at[pl.ds(r0, CHUNK)])

    return pl.kernel(body, out_type=out_type, mesh=mesh, scratch_types=scratch,
                     compiler_params=pltpu.CompilerParams(use_tc_tiling_on_sc=False))


_seg_sum_deg = _make_seg_sum(True)
_seg_sum = _make_seg_sum(False)


BR = 512
GRID = NP // BR


def _ln_val(t, g, b):
    mu = jnp.mean(t, axis=-1, keepdims=True)
    d = t - mu
    var = jnp.mean(d * d, axis=-1, keepdims=True)
    return d * lax.rsqrt(var + 1e-5) * g + b


def _leaky_val(t):
    return jnp.where(t >= 0, t, 0.01 * t)


def _row_spec(width):
    return pl.BlockSpec((BR, width), lambda i: (i, 0))


def _full_spec(shape):
    return pl.BlockSpec(shape, lambda i: (0,) * len(shape))


def _mm_ln(x, W, b, g, bt):
    K = x.shape[1]

    def body(x_ref, w_ref, b_ref, g_ref, bt_ref, o_ref):
        t = jnp.dot(x_ref[...], w_ref[...], preferred_element_type=_F32)
        o_ref[...] = _ln_val(t + b_ref[...], g_ref[...], bt_ref[...])

    return pl.pallas_call(
        body,
        grid=(GRID,),
        in_specs=[_row_spec(K), _full_spec((K, HID)), _full_spec((1, HID)),
                  _full_spec((1, HID)), _full_spec((1, HID))],
        out_specs=_row_spec(HID),
        out_shape=jax.ShapeDtypeStruct((NP, HID), _F32),
    )(x, W, b, g, bt)


def _mm_parts_split(parts, Ws):
    p = len(parts)

    def body(*refs):
        xs, ws = refs[:p], refs[p:2 * p]
        o_lo, o_hi = refs[2 * p], refs[2 * p + 1]
        t = jnp.dot(xs[0][...], ws[0][...], preferred_element_type=_F32)
        for i in range(1, p):
            t += jnp.dot(xs[i][...], ws[i][...], preferred_element_type=_F32)
        o_lo[...] = t[:, :128]
        o_hi[...] = t[:, 128:]

    return pl.pallas_call(
        body,
        grid=(GRID,),
        in_specs=[_row_spec(HID)] * p + [_full_spec((HID, HID))] * p,
        out_specs=[_row_spec(128), _row_spec(128)],
        out_shape=[jax.ShapeDtypeStruct((NP, 128), _F32)] * 2,
    )(*parts, *Ws)


def _post_layer(s_lo, s_hi, deg, hb, eW, eb, eg, ebt):
    def body(lo_ref, hi_ref, deg_ref, hb_ref, wlo_ref, whi_ref,
             eb_ref, g_ref, bt_ref, o_ref):
        t = jnp.dot(lo_ref[...], wlo_ref[...], preferred_element_type=_F32)
        t += jnp.dot(hi_ref[...], whi_ref[...], preferred_element_type=_F32)
        inv = 1.0 / jnp.maximum(deg_ref[:, 0:1], 1.0)
        hb = hb_ref[...]
        bias = (jnp.dot(hb[:, :128], wlo_ref[...], preferred_element_type=_F32)
                + jnp.dot(hb[:, 128:], whi_ref[...], preferred_element_type=_F32)
                + eb_ref[...])
        t = t * inv + bias
        o_ref[...] = _leaky_val(_ln_val(t, g_ref[...], bt_ref[...]))

    return pl.pallas_call(
        body,
        grid=(GRID,),
        in_specs=[_row_spec(128), _row_spec(128), _row_spec(16),
                  _full_spec((1, HID)), _full_spec((128, HID)),
                  _full_spec((128, HID)), _full_spec((1, HID)),
                  _full_spec((1, HID)), _full_spec((1, HID))],
        out_specs=_row_spec(HID),
        out_shape=jax.ShapeDtypeStruct((NP, HID), _F32),
    )(s_lo, s_hi, deg, hb, eW[:128], eW[128:], eb, eg, ebt)


def _head(parts, W3s, b3, g3, bt3, fcW_pad, fcb_pad):
    p = len(parts)

    def body(*refs):
        xs, ws = refs[:p], refs[p:2 * p]
        b_ref, g_ref, bt_ref, fw_ref, fb_ref, o_ref = refs[2 * p:]
        t = jnp.dot(xs[0][...], ws[0][...], preferred_element_type=_F32)
        for i in range(1, p):
            t += jnp.dot(xs[i][...], ws[i][...], preferred_element_type=_F32)
        z = _leaky_val(_ln_val(t + b_ref[...], g_ref[...], bt_ref[...]))
        o_ref[...] = jnp.dot(z, fw_ref[...], preferred_element_type=_F32) + fb_ref[...]

    return pl.pallas_call(
        body,
        grid=(GRID,),
        in_specs=[_row_spec(HID)] * p + [_full_spec((HID, HID))] * p
                 + [_full_spec((1, HID))] * 3
                 + [_full_spec((HID, 128)), _full_spec((1, 128))],
        out_specs=_row_spec(128),
        out_shape=jax.ShapeDtypeStruct((NP, 128), _F32),
    )(*parts, *W3s, b3, g3, bt3, fcW_pad, fcb_pad)


def kernel(x_all, edge_index, p, fc1_W, fc1_b, ln1_g, ln1_b,
           hW0, hb0, eW0, eb0, eg0, ebt0,
           hW1, hb1, eW1, eb1, eg1, ebt1,
           hW2, hb2, eW2, eb2, eg2, ebt2,
           fc3_W, fc3_b, ln3_g, ln3_b, fc_W, fc_b):
    del p
    hWs = [hW0, hW1, hW2]
    hbs = [hb0, hb1, hb2]
    eWs = [eW0, eW1, eW2]
    ebs = [eb0, eb1, eb2]
    egs = [eg0, eg1, eg2]
    ebts = [ebt0, ebt1, ebt2]

    row = lambda v: v.reshape(1, -1)

    xp = jnp.pad(x_all, ((0, NP - N), (0, 0)))

    npad = EP - E
    ar = jnp.arange(npad, dtype=jnp.int32)
    src = jnp.concatenate([edge_index[0], (ar * 37) % N])
    dst = jnp.concatenate([edge_index[1], N + (ar % 16)])

    x0 = _mm_ln(xp, fc1_W, row(fc1_b), row(ln1_g), row(ln1_b))

    parts = [x0]
    deg = None
    for i in range(LAYERS):
        Wsplit = [lax.slice(hWs[i], (j * HID, 0), ((j + 1) * HID, HID))
                  for j in range(i + 1)]
        y_lo, y_hi = _mm_parts_split(parts, Wsplit)
        if i == 0:
            s_lo, s_hi, deg = _seg_sum_deg(y_lo, y_hi, src, dst)
        else:
            s_lo, s_hi = _seg_sum(y_lo, y_hi, src, dst)
        h = _post_layer(s_lo, s_hi, deg, row(hbs[i]), eWs[i], row(ebs[i]),
                        row(egs[i]), row(ebts[i]))
        parts.append(h)

    W3s = [lax.slice(fc3_W, (j * HID, 0), ((j + 1) * HID, HID))
           for j in range(LAYERS + 1)]
    fcW_pad = jnp.pad(fc_W, ((0, 0), (0, 128 - NUM_CLASS)))
    fcb_pad = jnp.pad(fc_b, ((0, 128 - NUM_CLASS),))
    out = _head(parts, W3s, row(fc3_b), row(ln3_g), row(ln3_b),
                fcW_pad, row(fcb_pad))
    return out[:N, :NUM_CLASS]


# speedup vs baseline: 9.2665x; 1.5281x over previous
import functools

import jax
import jax.numpy as jnp
from jax import lax
from jax.experimental import pallas as pl
from jax.experimental.pallas import tpu as pltpu
from jax.experimental.pallas import tpu_sc as plsc

HID = 256
N = 10000
NP = 10240
E = 160000
EP = 163840
LAYERS = 3
NUM_CLASS = 2

NS = 16
NC = 2
CHUNK = 128
EPT = EP // NS
NCHUNK = EPT // CHUNK
RPT = NP // NS
RCH = RPT // CHUNK

_F32 = jnp.float32


NB = 4
QW = 64


def _make_seg_sum():
    mesh = plsc.VectorSubcoreMesh(core_axis_name="c", subcore_axis_name="s")

    out_type = [jax.ShapeDtypeStruct((NP, QW), _F32)] * 4
    scratch = [
        pltpu.VMEM((NCHUNK, CHUNK), jnp.int32),
        pltpu.VMEM((NCHUNK, CHUNK), jnp.int32),
        pltpu.VMEM((NB, CHUNK, QW), _F32),
        pltpu.VMEM((CHUNK, QW), _F32),
        pltpu.VMEM_SHARED((NP, QW), _F32),
    ] + [pltpu.SemaphoreType.DMA] * NB

    def body(yq0, yq1, yq2, yq3, src3, dst3, o0, o1, o2, o3,
             srcb, dstb, rows, zbuf, acc, *sems):
        yqs = [[yq0, yq1], [yq2, yq3]]
        outs = [[o0, o1], [o2, o3]]

        c = lax.axis_index("c")
        s = lax.axis_index("s")
        base = s * RPT

        zeros16 = jnp.zeros((16,), _F32)

        def _zero_row(r, _):
            def _zero_col(cc, _):
                zbuf[r, pl.ds(cc * 16, 16)] = zeros16
                return 0
            return lax.fori_loop(0, QW // 16, _zero_col, 0)

        lax.fori_loop(0, CHUNK, _zero_row, 0)

        pltpu.sync_copy(src3.at[s], srcb)
        pltpu.sync_copy(dst3.at[s], dstb)

        def fire(p, kk, b):
            @pl.when(c == 0)
            def _():
                pltpu.async_copy(yqs[0][p].at[srcb.at[kk]], rows.at[b], sems[b])

            @pl.when(c == 1)
            def _():
                pltpu.async_copy(yqs[1][p].at[srcb.at[kk]], rows.at[b], sems[b])

        def drain(p, kk, b):
            pltpu.make_async_copy(yqs[0][p].at[srcb.at[kk]], rows.at[b],
                                  sems[b]).wait()
            pltpu.sync_copy(rows.at[b], acc.at[dstb.at[kk]], add=True)

        for p in range(2):
            for j in range(RCH):
                pltpu.sync_copy(zbuf, acc.at[pl.ds(base + j * CHUNK, CHUNK)])
            plsc.subcore_barrier()

            for b in range(NB):
                fire(p, b, b)

            def steady(t, _):
                for b in range(NB):
                    kk = t * NB + b
                    drain(p, kk, b)
                    fire(p, kk + NB, b)
                return 0

            lax.fori_loop(0, (NCHUNK - NB) // NB, steady, 0)
            for b in range(NB):
                drain(p, NCHUNK - NB + b, b)

            plsc.subcore_barrier()

            for j in range(RCH):
                r0 = base + j * CHUNK
                pltpu.sync_copy(acc.at[pl.ds(r0, CHUNK)], rows.at[0])

                @pl.when(c == 0)
                def _():
                    pltpu.sync_copy(rows.at[0], outs[0][p].at[pl.ds(r0, CHUNK)])

                @pl.when(c == 1)
                def _():
                    pltpu.sync_copy(rows.at[0], outs[1][p].at[pl.ds(r0, CHUNK)])

    return pl.kernel(body, out_type=out_type, mesh=mesh, scratch_types=scratch,
                     compiler_params=pltpu.CompilerParams(use_tc_tiling_on_sc=False))


def _make_deg():
    mesh = plsc.VectorSubcoreMesh(core_axis_name="c", subcore_axis_name="s")
    scratch = [
        pltpu.VMEM((NCHUNK, CHUNK), jnp.int32),
        pltpu.VMEM((CHUNK, 16), _F32),
        pltpu.VMEM((CHUNK, 16), _F32),
        pltpu.VMEM_SHARED((NP, 16), _F32),
    ]

    def body(dst3, deg_out, dstb, ones_v, zb, deg_acc):
        c = lax.axis_index("c")
        s = lax.axis_index("s")
        base = s * RPT
        zeros16 = jnp.zeros((16,), _F32)
        ones16 = jnp.ones((16,), _F32)

        def _fill(r, _):
            ones_v[r, :] = ones16
            zb[r, :] = zeros16
            return 0

        lax.fori_loop(0, CHUNK, _fill, 0)
        pltpu.sync_copy(dst3.at[s], dstb)
        for j in range(RCH):
            pltpu.sync_copy(zb, deg_acc.at[pl.ds(base + j * CHUNK, CHUNK)])
        plsc.subcore_barrier()

        @pl.when(c == 0)
        def _():
            def step(k, _):
                pltpu.sync_copy(ones_v, deg_acc.at[dstb.at[k]], add=True)
                return 0
            lax.fori_loop(0, NCHUNK, step, 0)

        plsc.subcore_barrier()

        @pl.when(c == 0)
        def _():
            for j in range(RCH):
                r0 = base + j * CHUNK
                pltpu.sync_copy(deg_acc.at[pl.ds(r0, CHUNK)], zb)
                pltpu.sync_copy(zb, deg_out.at[pl.ds(r0, CHUNK)])

    return pl.kernel(body, out_type=jax.ShapeDtypeStruct((NP, 16), _F32),
                     mesh=mesh, scratch_types=scratch,
                     compiler_params=pltpu.CompilerParams(use_tc_tiling_on_sc=False))


_seg_sum = _make_seg_sum()
_deg = _make_deg()


BR = 512
GRID = NP // BR


def _ln_val(t, g, b):
    mu = jnp.mean(t, axis=-1, keepdims=True)
    d = t - mu
    var = jnp.mean(d * d, axis=-1, keepdims=True)
    return d * lax.rsqrt(var + 1e-5) * g + b


def _leaky_val(t):
    return jnp.where(t >= 0, t, 0.01 * t)


def _row_spec(width):
    return pl.BlockSpec((BR, width), lambda i: (i, 0))


def _full_spec(shape):
    return pl.BlockSpec(shape, lambda i: (0,) * len(shape))


def _mm_ln(x, W, b, g, bt):
    K = x.shape[1]

    def body(x_ref, w_ref, b_ref, g_ref, bt_ref, o_ref):
        t = jnp.dot(x_ref[...], w_ref[...], preferred_element_type=_F32)
        o_ref[...] = _ln_val(t + b_ref[...], g_ref[...], bt_ref[...])

    return pl.pallas_call(
        body,
        grid=(GRID,),
        in_specs=[_row_spec(K), _full_spec((K, HID)), _full_spec((1, HID)),
                  _full_spec((1, HID)), _full_spec((1, HID))],
        out_specs=_row_spec(HID),
        out_shape=jax.ShapeDtypeStruct((NP, HID), _F32),
    )(x, W, b, g, bt)


def _mm_parts_split(parts, Ws):
    p = len(parts)

    def body(*refs):
        xs, ws = refs[:p], refs[p:2 * p]
        os = refs[2 * p:]
        t = jnp.dot(xs[0][...], ws[0][...], preferred_element_type=_F32)
        for i in range(1, p):
            t += jnp.dot(xs[i][...], ws[i][...], preferred_element_type=_F32)
        for q in range(4):
            os[q][...] = t[:, q * QW:(q + 1) * QW]

    return pl.pallas_call(
        body,
        grid=(GRID,),
        in_specs=[_row_spec(HID)] * p + [_full_spec((HID, HID))] * p,
        out_specs=[_row_spec(QW)] * 4,
        out_shape=[jax.ShapeDtypeStruct((NP, QW), _F32)] * 4,
    )(*parts, *Ws)


def _post_layer(sqs, deg, hb, eW, eb, eg, ebt):
    def body(s0_ref, s1_ref, s2_ref, s3_ref, deg_ref, hb_ref,
             w0_ref, w1_ref, w2_ref, w3_ref,
             eb_ref, g_ref, bt_ref, o_ref):
        s_refs = [s0_ref, s1_ref, s2_ref, s3_ref]
        w_refs = [w0_ref, w1_ref, w2_ref, w3_ref]
        hb = hb_ref[...]
        t = jnp.dot(s_refs[0][...], w_refs[0][...], preferred_element_type=_F32)
        bias = jnp.dot(hb[:, :QW], w_refs[0][...], preferred_element_type=_F32)
        for q in range(1, 4):
            t += jnp.dot(s_refs[q][...], w_refs[q][...],
                         preferred_element_type=_F32)
            bias += jnp.dot(hb[:, q * QW:(q + 1) * QW], w_refs[q][...],
                            preferred_element_type=_F32)
        inv = 1.0 / jnp.maximum(deg_ref[:, 0:1], 1.0)
        t = t * inv + bias + eb_ref[...]
        o_ref[...] = _leaky_val(_ln_val(t, g_ref[...], bt_ref[...]))

    return pl.pallas_call(
        body,
        grid=(GRID,),
        in_specs=[_row_spec(QW)] * 4 + [_row_spec(16), _full_spec((1, HID))]
                 + [_full_spec((QW, HID))] * 4
                 + [_full_spec((1, HID))] * 3,
        out_specs=_row_spec(HID),
        out_shape=jax.ShapeDtypeStruct((NP, HID), _F32),
    )(*sqs, deg, hb, *(eW[q * QW:(q + 1) * QW] for q in range(4)),
      eb, eg, ebt)


def _head(parts, W3s, b3, g3, bt3, fcW_pad, fcb_pad):
    p = len(parts)

    def body(*refs):
        xs, ws = refs[:p], refs[p:2 * p]
        b_ref, g_ref, bt_ref, fw_ref, fb_ref, o_ref = refs[2 * p:]
        t = jnp.dot(xs[0][...], ws[0][...], preferred_element_type=_F32)
        for i in range(1, p):
            t += jnp.dot(xs[i][...], ws[i][...], preferred_element_type=_F32)
        z = _leaky_val(_ln_val(t + b_ref[...], g_ref[...], bt_ref[...]))
        o_ref[...] = jnp.dot(z, fw_ref[...], preferred_element_type=_F32) + fb_ref[...]

    return pl.pallas_call(
        body,
        grid=(GRID,),
        in_specs=[_row_spec(HID)] * p + [_full_spec((HID, HID))] * p
                 + [_full_spec((1, HID))] * 3
                 + [_full_spec((HID, 128)), _full_spec((1, 128))],
        out_specs=_row_spec(128),
        out_shape=jax.ShapeDtypeStruct((NP, 128), _F32),
    )(*parts, *W3s, b3, g3, bt3, fcW_pad, fcb_pad)


def kernel(x_all, edge_index, p, fc1_W, fc1_b, ln1_g, ln1_b,
           hW0, hb0, eW0, eb0, eg0, ebt0,
           hW1, hb1, eW1, eb1, eg1, ebt1,
           hW2, hb2, eW2, eb2, eg2, ebt2,
           fc3_W, fc3_b, ln3_g, ln3_b, fc_W, fc_b):
    del p
    hWs = [hW0, hW1, hW2]
    hbs = [hb0, hb1, hb2]
    eWs = [eW0, eW1, eW2]
    ebs = [eb0, eb1, eb2]
    egs = [eg0, eg1, eg2]
    ebts = [ebt0, ebt1, ebt2]

    row = lambda v: v.reshape(1, -1)

    xp = jnp.pad(x_all, ((0, NP - N), (0, 0)))

    npad = EP - E
    ar = jnp.arange(npad, dtype=jnp.int32)
    src = jnp.concatenate([edge_index[0], (ar * 37) % N]).reshape(NS, NCHUNK, CHUNK)
    dst = jnp.concatenate([edge_index[1], N + (ar % 16)]).reshape(NS, NCHUNK, CHUNK)

    deg = _deg(dst)
    x0 = _mm_ln(xp, fc1_W, row(fc1_b), row(ln1_g), row(ln1_b))

    parts = [x0]
    for i in range(LAYERS):
        Wsplit = [lax.slice(hWs[i], (j * HID, 0), ((j + 1) * HID, HID))
                  for j in range(i + 1)]
        yqs = _mm_parts_split(parts, Wsplit)
        sqs = _seg_sum(*yqs, src, dst)
        h = _post_layer(sqs, deg, row(hbs[i]), eWs[i], row(ebs[i]),
                        row(egs[i]), row(ebts[i]))
        parts.append(h)

    W3s = [lax.slice(fc3_W, (j * HID, 0), ((j + 1) * HID, HID))
           for j in range(LAYERS + 1)]
    fcW_pad = jnp.pad(fc_W, ((0, 0), (0, 128 - NUM_CLASS)))
    fcb_pad = jnp.pad(fc_b, ((0, 128 - NUM_CLASS),))
    out = _head(parts, W3s, row(fc3_b), row(ln3_g), row(ln3_b),
                fcW_pad, row(fcb_pad))
    return out[:N, :NUM_CLASS]


# speedup vs baseline: 10.1473x; 1.0951x over previous
import functools

import jax
import jax.numpy as jnp
from jax import lax
from jax.experimental import pallas as pl
from jax.experimental.pallas import tpu as pltpu
from jax.experimental.pallas import tpu_sc as plsc

HID = 256
N = 10000
NP = 10240
E = 160000
EP = 163840
LAYERS = 3
NUM_CLASS = 2

NS = 16
NC = 2
CHUNK = 128
EPT = EP // NS
NCHUNK = EPT // CHUNK
RPT = NP // NS
RCH = RPT // CHUNK

_F32 = jnp.float32


NB = 4
QW = 64


def _make_seg_sum():
    mesh = plsc.VectorSubcoreMesh(core_axis_name="c", subcore_axis_name="s")

    out_type = [jax.ShapeDtypeStruct((NP, QW), _F32)] * 4
    scratch = [
        pltpu.VMEM((NCHUNK, CHUNK), jnp.int32),
        pltpu.VMEM((NCHUNK, CHUNK), jnp.int32),
        pltpu.VMEM((NB, CHUNK, QW), _F32),
        pltpu.VMEM((CHUNK, QW), _F32),
        pltpu.VMEM_SHARED((NP, QW), _F32),
    ] + [pltpu.SemaphoreType.DMA] * NB

    def body(yq0, yq1, yq2, yq3, src3, dst3, o0, o1, o2, o3,
             srcb, dstb, rows, zbuf, acc, *sems):
        yqs = [[yq0, yq1], [yq2, yq3]]
        outs = [[o0, o1], [o2, o3]]

        c = lax.axis_index("c")
        s = lax.axis_index("s")
        base = s * RPT

        zeros16 = jnp.zeros((16,), _F32)

        def _zero_row(r, _):
            def _zero_col(cc, _):
                zbuf[r, pl.ds(cc * 16, 16)] = zeros16
                return 0
            return lax.fori_loop(0, QW // 16, _zero_col, 0)

        lax.fori_loop(0, CHUNK, _zero_row, 0)

        pltpu.sync_copy(src3.at[s], srcb)
        pltpu.sync_copy(dst3.at[s], dstb)

        def fire(p, kk, b):
            @pl.when(c == 0)
            def _():
                pltpu.async_copy(yqs[0][p].at[srcb.at[kk]], rows.at[b], sems[b])

            @pl.when(c == 1)
            def _():
                pltpu.async_copy(yqs[1][p].at[srcb.at[kk]], rows.at[b], sems[b])

        def drain(p, kk, b):
            pltpu.make_async_copy(yqs[0][p].at[srcb.at[kk]], rows.at[b],
                                  sems[b]).wait()
            pltpu.sync_copy(rows.at[b], acc.at[dstb.at[kk]], add=True)

        for p in range(2):
            for j in range(RCH):
                pltpu.sync_copy(zbuf, acc.at[pl.ds(base + j * CHUNK, CHUNK)])
            plsc.subcore_barrier()

            for b in range(NB):
                fire(p, b, b)

            def steady(t, _):
                for b in range(NB):
                    kk = t * NB + b
                    drain(p, kk, b)
                    fire(p, kk + NB, b)
                return 0

            lax.fori_loop(0, (NCHUNK - NB) // NB, steady, 0)
            for b in range(NB):
                drain(p, NCHUNK - NB + b, b)

            plsc.subcore_barrier()

            for j in range(RCH):
                r0 = base + j * CHUNK
                pltpu.sync_copy(acc.at[pl.ds(r0, CHUNK)], rows.at[0])

                @pl.when(c == 0)
                def _():
                    pltpu.sync_copy(rows.at[0], outs[0][p].at[pl.ds(r0, CHUNK)])

                @pl.when(c == 1)
                def _():
                    pltpu.sync_copy(rows.at[0], outs[1][p].at[pl.ds(r0, CHUNK)])

    return pl.kernel(body, out_type=out_type, mesh=mesh, scratch_types=scratch,
                     compiler_params=pltpu.CompilerParams(use_tc_tiling_on_sc=False))


def _make_deg():
    mesh = plsc.VectorSubcoreMesh(core_axis_name="c", subcore_axis_name="s")
    scratch = [
        pltpu.VMEM((NCHUNK, CHUNK), jnp.int32),
        pltpu.VMEM((CHUNK, 16), _F32),
        pltpu.VMEM((CHUNK, 16), _F32),
        pltpu.VMEM_SHARED((NP, 16), _F32),
    ]

    def body(dst3, deg_out, dstb, ones_v, zb, deg_acc):
        c = lax.axis_index("c")
        s = lax.axis_index("s")
        base = s * RPT
        zeros16 = jnp.zeros((16,), _F32)
        ones16 = jnp.ones((16,), _F32)

        def _fill(r, _):
            ones_v[r, :] = ones16
            zb[r, :] = zeros16
            return 0

        lax.fori_loop(0, CHUNK, _fill, 0)
        pltpu.sync_copy(dst3.at[s], dstb)
        for j in range(RCH):
            pltpu.sync_copy(zb, deg_acc.at[pl.ds(base + j * CHUNK, CHUNK)])
        plsc.subcore_barrier()

        @pl.when(c == 0)
        def _():
            def step(k, _):
                pltpu.sync_copy(ones_v, deg_acc.at[dstb.at[k]], add=True)
                return 0
            lax.fori_loop(0, NCHUNK, step, 0)

        plsc.subcore_barrier()

        @pl.when(c == 0)
        def _():
            for j in range(RCH):
                r0 = base + j * CHUNK
                pltpu.sync_copy(deg_acc.at[pl.ds(r0, CHUNK)], zb)
                pltpu.sync_copy(zb, deg_out.at[pl.ds(r0, CHUNK)])

    return pl.kernel(body, out_type=jax.ShapeDtypeStruct((NP, 16), _F32),
                     mesh=mesh, scratch_types=scratch,
                     compiler_params=pltpu.CompilerParams(use_tc_tiling_on_sc=False))


_seg_sum = _make_seg_sum()
_deg = _make_deg()


BR = 512
GRID = NP // BR


def _ln_val(t, g, b):
    mu = jnp.mean(t, axis=-1, keepdims=True)
    d = t - mu
    var = jnp.mean(d * d, axis=-1, keepdims=True)
    return d * lax.rsqrt(var + 1e-5) * g + b


def _leaky_val(t):
    return jnp.where(t >= 0, t, 0.01 * t)


def _row_spec(width):
    return pl.BlockSpec((BR, width), lambda i: (i, 0))


def _full_spec(shape):
    return pl.BlockSpec(shape, lambda i: (0,) * len(shape))


def _mm_ln_split(x, W, b, g, bt, hW):
    K = x.shape[1]

    def body(x_ref, w_ref, b_ref, g_ref, bt_ref, hw_ref, o_ref, *oq):
        t = jnp.dot(x_ref[...], w_ref[...], preferred_element_type=_F32)
        x0 = _ln_val(t + b_ref[...], g_ref[...], bt_ref[...])
        o_ref[...] = x0
        y = jnp.dot(x0, hw_ref[...], preferred_element_type=_F32)
        for q in range(4):
            oq[q][...] = y[:, q * QW:(q + 1) * QW]

    return pl.pallas_call(
        body,
        grid=(GRID,),
        in_specs=[_row_spec(K), _full_spec((K, HID)), _full_spec((1, HID)),
                  _full_spec((1, HID)), _full_spec((1, HID)),
                  _full_spec((HID, HID))],
        out_specs=[_row_spec(HID)] + [_row_spec(QW)] * 4,
        out_shape=[jax.ShapeDtypeStruct((NP, HID), _F32)]
                  + [jax.ShapeDtypeStruct((NP, QW), _F32)] * 4,
    )(x, W, b, g, bt, hW)


def _post_val(s_vals, deg_ref, hb_ref, w_refs, eb_ref, g_ref, bt_ref):
    hb = hb_ref[...]
    t = jnp.dot(s_vals[0], w_refs[0][...], preferred_element_type=_F32)
    bias = jnp.dot(hb[:, :QW], w_refs[0][...], preferred_element_type=_F32)
    for q in range(1, 4):
        t += jnp.dot(s_vals[q], w_refs[q][...], preferred_element_type=_F32)
        bias += jnp.dot(hb[:, q * QW:(q + 1) * QW], w_refs[q][...],
                        preferred_element_type=_F32)
    inv = 1.0 / jnp.maximum(deg_ref[:, 0:1], 1.0)
    t = t * inv + bias + eb_ref[...]
    return _leaky_val(_ln_val(t, g_ref[...], bt_ref[...]))


def _post_mm(sqs, deg, hb, eW, eb, eg, ebt, parts, hWn):
    np_ = len(parts)

    def body(*refs):
        i = 0
        s_refs = refs[i:i + 4]; i += 4
        deg_ref = refs[i]; i += 1
        hb_ref = refs[i]; i += 1
        ew_refs = refs[i:i + 4]; i += 4
        eb_ref, g_ref, bt_ref = refs[i:i + 3]; i += 3
        part_refs = refs[i:i + np_]; i += np_
        hw_refs = refs[i:i + np_ + 1]; i += np_ + 1
        h_ref = refs[i]; i += 1
        oq = refs[i:]

        h = _post_val([r[...] for r in s_refs], deg_ref, hb_ref, ew_refs,
                      eb_ref, g_ref, bt_ref)
        h_ref[...] = h
        y = jnp.dot(h, hw_refs[np_][...], preferred_element_type=_F32)
        for j in range(np_):
            y += jnp.dot(part_refs[j][...], hw_refs[j][...],
                         preferred_element_type=_F32)
        for q in range(4):
            oq[q][...] = y[:, q * QW:(q + 1) * QW]

    return pl.pallas_call(
        body,
        grid=(GRID,),
        in_specs=[_row_spec(QW)] * 4 + [_row_spec(16), _full_spec((1, HID))]
                 + [_full_spec((QW, HID))] * 4 + [_full_spec((1, HID))] * 3
                 + [_row_spec(HID)] * np_ + [_full_spec((HID, HID))] * (np_ + 1),
        out_specs=[_row_spec(HID)] + [_row_spec(QW)] * 4,
        out_shape=[jax.ShapeDtypeStruct((NP, HID), _F32)]
                  + [jax.ShapeDtypeStruct((NP, QW), _F32)] * 4,
    )(*sqs, deg, hb, *(eW[q * QW:(q + 1) * QW] for q in range(4)),
      eb, eg, ebt, *parts,
      *(lax.slice(hWn, (j * HID, 0), ((j + 1) * HID, HID))
        for j in range(np_ + 1)))


def _post_head(sqs, deg, hb, eW, eb, eg, ebt, parts,
               W3s, b3, g3, bt3, fcW_pad, fcb_pad):
    np_ = len(parts)

    def body(*refs):
        i = 0
        s_refs = refs[i:i + 4]; i += 4
        deg_ref = refs[i]; i += 1
        hb_ref = refs[i]; i += 1
        ew_refs = refs[i:i + 4]; i += 4
        eb_ref, g_ref, bt_ref = refs[i:i + 3]; i += 3
        part_refs = refs[i:i + np_]; i += np_
        w3_refs = refs[i:i + np_ + 1]; i += np_ + 1
        b3_ref, g3_ref, bt3_ref, fw_ref, fb_ref = refs[i:i + 5]; i += 5
        o_ref = refs[i]

        h = _post_val([r[...] for r in s_refs], deg_ref, hb_ref, ew_refs,
                      eb_ref, g_ref, bt_ref)
        t = jnp.dot(h, w3_refs[np_][...], preferred_element_type=_F32)
        for j in range(np_):
            t += jnp.dot(part_refs[j][...], w3_refs[j][...],
                         preferred_element_type=_F32)
        z = _leaky_val(_ln_val(t + b3_ref[...], g3_ref[...], bt3_ref[...]))
        o_ref[...] = jnp.dot(z, fw_ref[...], preferred_element_type=_F32) \
            + fb_ref[...]

    return pl.pallas_call(
        body,
        grid=(GRID,),
        in_specs=[_row_spec(QW)] * 4 + [_row_spec(16), _full_spec((1, HID))]
                 + [_full_spec((QW, HID))] * 4 + [_full_spec((1, HID))] * 3
                 + [_row_spec(HID)] * np_ + [_full_spec((HID, HID))] * (np_ + 1)
                 + [_full_spec((1, HID))] * 3
                 + [_full_spec((HID, 128)), _full_spec((1, 128))],
        out_specs=_row_spec(128),
        out_shape=jax.ShapeDtypeStruct((NP, 128), _F32),
    )(*sqs, deg, hb, *(eW[q * QW:(q + 1) * QW] for q in range(4)),
      eb, eg, ebt, *parts, *W3s, b3, g3, bt3, fcW_pad, fcb_pad)


def kernel(x_all, edge_index, p, fc1_W, fc1_b, ln1_g, ln1_b,
           hW0, hb0, eW0, eb0, eg0, ebt0,
           hW1, hb1, eW1, eb1, eg1, ebt1,
           hW2, hb2, eW2, eb2, eg2, ebt2,
           fc3_W, fc3_b, ln3_g, ln3_b, fc_W, fc_b):
    del p
    hWs = [hW0, hW1, hW2]
    hbs = [hb0, hb1, hb2]
    eWs = [eW0, eW1, eW2]
    ebs = [eb0, eb1, eb2]
    egs = [eg0, eg1, eg2]
    ebts = [ebt0, ebt1, ebt2]

    row = lambda v: v.reshape(1, -1)

    xp = jnp.pad(x_all, ((0, NP - N), (0, 0)))

    npad = EP - E
    ar = jnp.arange(npad, dtype=jnp.int32)
    src = jnp.concatenate([edge_index[0], (ar * 37) % N]).reshape(NS, NCHUNK, CHUNK)
    dst = jnp.concatenate([edge_index[1], N + (ar % 16)]).reshape(NS, NCHUNK, CHUNK)

    deg = _deg(dst)
    x0, *yqs = _mm_ln_split(xp, fc1_W, row(fc1_b), row(ln1_g), row(ln1_b), hW0)

    parts = [x0]
    for i in range(LAYERS - 1):
        sqs = _seg_sum(*yqs, src, dst)
        h, *yqs = _post_mm(sqs, deg, row(hbs[i]), eWs[i], row(ebs[i]),
                           row(egs[i]), row(ebts[i]), parts, hWs[i + 1])
        parts.append(h)

    sqs = _seg_sum(*yqs, src, dst)
    W3s = [lax.slice(fc3_W, (j * HID, 0), ((j + 1) * HID, HID))
           for j in range(LAYERS + 1)]
    fcW_pad = jnp.pad(fc_W, ((0, 0), (0, 128 - NUM_CLASS)))
    fcb_pad = jnp.pad(fc_b, ((0, 128 - NUM_CLASS),))
    out = _post_head(sqs, deg, row(hbs[2]), eWs[2], row(ebs[2]),
                     row(egs[2]), row(ebts[2]), parts,
                     W3s, row(fc3_b), row(ln3_g), row(ln3_b),
                     fcW_pad, row(fcb_pad))
    return out[:N, :NUM_CLASS]


# speedup vs baseline: 12.3281x; 1.2149x over previous
import jax
import jax.numpy as jnp
from jax import lax
from jax.experimental import pallas as pl
from jax.experimental.pallas import tpu as pltpu
from jax.experimental.pallas import tpu_sc as plsc

HID = 256
N = 10000
NP = 10240
E = 160000
EP = 163840
LAYERS = 3
NUM_CLASS = 2

NS = 16
CHUNK = 128
NCHUNK = EP // NS // CHUNK
RPT = NP // NS
RCH = RPT // CHUNK
NB = 2
SCH = 16
NSTAGE = NCHUNK // SCH

_F32 = jnp.float32

_SC_PARAMS = pltpu.CompilerParams(use_tc_tiling_on_sc=True)


def _make_seg_sum():
    mesh = plsc.VectorSubcoreMesh(core_axis_name="c", subcore_axis_name="s")

    out_type = [jax.ShapeDtypeStruct((NP, 128), _F32)] * 2
    scratch = [
        pltpu.VMEM((NB, SCH, CHUNK), jnp.int32),
        pltpu.VMEM((NB, SCH, CHUNK), jnp.int32),
        pltpu.VMEM((NB, CHUNK, 128), _F32),
        pltpu.VMEM_SHARED((NP, 128), _F32),
        pltpu.SemaphoreType.DMA,
    ] + [pltpu.SemaphoreType.DMA] * NB
    def body(y_lo, y_hi, src3, dst3, out_lo, out_hi,
             srcq, dstq, rows, acc, isem, g0, g1):
        sems = [g0, g1]
        c = lax.axis_index("c")
        s = lax.axis_index("s")
        base = s * RPT

        zeros16 = jnp.zeros((16,), _F32)

        def _zero_row(r, _):
            def _zero_col(cc, _):
                rows[0, r, pl.ds(cc * 16, 16)] = zeros16
                return 0
            return lax.fori_loop(0, 128 // 16, _zero_col, 0)

        lax.fori_loop(0, CHUNK, _zero_row, 0)
        for j in range(RCH):
            pltpu.sync_copy(rows.at[0], acc.at[pl.ds(base + j * CHUNK, CHUNK)])

        def stage_copy(q, ib):
            r0 = q * SCH
            return [pltpu.async_copy(src3.at[s, pl.ds(r0, SCH)], srcq.at[ib],
                                     isem),
                    pltpu.async_copy(dst3.at[s, pl.ds(r0, SCH)], dstq.at[ib],
                                     isem)]

        for d in stage_copy(0, 0):
            d.wait()
        plsc.subcore_barrier()

        def fire(ib, j, b):
            @pl.when(c == 0)
            def _():
                pltpu.async_copy(y_lo.at[srcq.at[ib, j]], rows.at[b], sems[b])

            @pl.when(c == 1)
            def _():
                pltpu.async_copy(y_hi.at[srcq.at[ib, j]], rows.at[b], sems[b])

        def drain(ib, j, b):
            pltpu.make_async_copy(y_lo.at[srcq.at[ib, j]], rows.at[b],
                                  sems[b]).wait()
            pltpu.sync_copy(rows.at[b], acc.at[dstq.at[ib, j]], add=True)

        for q in range(NSTAGE):
            ib = q % 2
            nxt = stage_copy(q + 1, 1 - ib) if q + 1 < NSTAGE else []
            for b in range(NB):
                fire(ib, b, b)

            def steady(t, _):
                for b in range(NB):
                    j = t * NB + b
                    drain(ib, j, b)
                    fire(ib, j + NB, b)
                return 0

            lax.fori_loop(0, (SCH - NB) // NB, steady, 0)
            for b in range(NB):
                drain(ib, SCH - NB + b, b)
            for d in nxt:
                d.wait()

        plsc.subcore_barrier()

        for j in range(RCH):
            r0 = base + j * CHUNK
            pltpu.sync_copy(acc.at[pl.ds(r0, CHUNK)], rows.at[0])

            @pl.when(c == 0)
            def _():
                pltpu.sync_copy(rows.at[0], out_lo.at[pl.ds(r0, CHUNK)])

            @pl.when(c == 1)
            def _():
                pltpu.sync_copy(rows.at[0], out_hi.at[pl.ds(r0, CHUNK)])

    return pl.kernel(body, out_type=out_type, mesh=mesh, scratch_types=scratch,
                     compiler_params=_SC_PARAMS)


def _make_deg():
    mesh = plsc.VectorSubcoreMesh(core_axis_name="c", subcore_axis_name="s")
    scratch = [
        pltpu.VMEM((NCHUNK, CHUNK), jnp.int32),
        pltpu.VMEM((CHUNK, 16), _F32),
        pltpu.VMEM((CHUNK, 16), _F32),
        pltpu.VMEM_SHARED((NP, 16), _F32),
    ]

    def body(dst3, deg_out, dstb, ones_v, zb, deg_acc):
        c = lax.axis_index("c")
        s = lax.axis_index("s")
        base = s * RPT
        zeros16 = jnp.zeros((16,), _F32)
        ones16 = jnp.ones((16,), _F32)

        def _fill(r, _):
            ones_v[r, :] = ones16
            zb[r, :] = zeros16
            return 0

        lax.fori_loop(0, CHUNK, _fill, 0)
        pltpu.sync_copy(dst3.at[s], dstb)
        for j in range(RCH):
            pltpu.sync_copy(zb, deg_acc.at[pl.ds(base + j * CHUNK, CHUNK)])
        plsc.subcore_barrier()

        @pl.when(c == 0)
        def _():
            def step(k, _):
                pltpu.sync_copy(ones_v, deg_acc.at[dstb.at[k]], add=True)
                return 0
            lax.fori_loop(0, NCHUNK, step, 0)

        plsc.subcore_barrier()

        @pl.when(c == 0)
        def _():
            for j in range(RCH):
                r0 = base + j * CHUNK
                pltpu.sync_copy(deg_acc.at[pl.ds(r0, CHUNK)], zb)
                pltpu.sync_copy(zb, deg_out.at[pl.ds(r0, CHUNK)])

    return pl.kernel(body, out_type=jax.ShapeDtypeStruct((NP, 16), _F32),
                     mesh=mesh, scratch_types=scratch,
                     compiler_params=_SC_PARAMS)


_seg_sum = _make_seg_sum()
_deg = _make_deg()


BR = 1000
GRID = N // BR


def _ln_val(t, g, b):
    mu = jnp.mean(t, axis=-1, keepdims=True)
    d = t - mu
    var = jnp.mean(d * d, axis=-1, keepdims=True)
    return d * lax.rsqrt(var + 1e-5) * g + b


def _leaky_val(t):
    return jnp.where(t >= 0, t, 0.01 * t)


def _row_spec(width):
    return pl.BlockSpec((BR, width), lambda i: (i, 0))


def _full_spec(shape):
    return pl.BlockSpec(shape, lambda i: (0,) * len(shape))


def _halves(y, os):
    os[0][...] = y[:, :128]
    os[1][...] = y[:, 128:]


def _mm_ln_split(x, W, b, g, bt, hW):
    K = x.shape[1]

    def body(x_ref, w_ref, b_ref, g_ref, bt_ref, hw_ref, o_ref, *oh):
        t = jnp.dot(x_ref[...], w_ref[...], preferred_element_type=_F32)
        x0 = _ln_val(t + b_ref[...], g_ref[...], bt_ref[...])
        o_ref[...] = x0
        _halves(jnp.dot(x0, hw_ref[...], preferred_element_type=_F32), oh)

    return pl.pallas_call(
        body,
        grid=(GRID,),
        in_specs=[_row_spec(K), _full_spec((K, HID)), _full_spec((1, HID)),
                  _full_spec((1, HID)), _full_spec((1, HID)),
                  _full_spec((HID, HID))],
        out_specs=[_row_spec(HID)] + [_row_spec(128)] * 2,
        out_shape=[jax.ShapeDtypeStruct((N, HID), _F32)]
                  + [jax.ShapeDtypeStruct((N, 128), _F32)] * 2,
    )(x, W, b, g, bt, hW)


def _post_val(s_vals, deg_ref, hb_ref, w_refs, eb_ref, g_ref, bt_ref):
    hb = hb_ref[...]
    t = jnp.dot(s_vals[0], w_refs[0][...], preferred_element_type=_F32)
    bias = jnp.dot(hb[:, :128], w_refs[0][...], preferred_element_type=_F32)
    t += jnp.dot(s_vals[1], w_refs[1][...], preferred_element_type=_F32)
    bias += jnp.dot(hb[:, 128:], w_refs[1][...], preferred_element_type=_F32)
    inv = 1.0 / jnp.maximum(deg_ref[:, 0:1], 1.0)
    t = t * inv + bias + eb_ref[...]
    return _leaky_val(_ln_val(t, g_ref[...], bt_ref[...]))


_POST_SPECS = ([_row_spec(128)] * 2 + [_row_spec(16), _full_spec((1, HID))]
               + [_full_spec((128, HID))] * 2 + [_full_spec((1, HID))] * 3)


def _post_mm(shs, deg, hb, eW, eb, eg, ebt, parts, hWn):
    np_ = len(parts)

    def body(*refs):
        i = 0
        s_refs = refs[i:i + 2]; i += 2
        deg_ref, hb_ref = refs[i:i + 2]; i += 2
        ew_refs = refs[i:i + 2]; i += 2
        eb_ref, g_ref, bt_ref = refs[i:i + 3]; i += 3
        part_refs = refs[i:i + np_]; i += np_
        hw_refs = refs[i:i + np_ + 1]; i += np_ + 1
        h_ref = refs[i]; i += 1
        oh = refs[i:]

        h = _post_val([r[...] for r in s_refs], deg_ref, hb_ref, ew_refs,
                      eb_ref, g_ref, bt_ref)
        h_ref[...] = h
        y = jnp.dot(h, hw_refs[np_][...], preferred_element_type=_F32)
        for j in range(np_):
            y += jnp.dot(part_refs[j][...], hw_refs[j][...],
                         preferred_element_type=_F32)
        _halves(y, oh)

    return pl.pallas_call(
        body,
        grid=(GRID,),
        in_specs=_POST_SPECS
                 + [_row_spec(HID)] * np_ + [_full_spec((HID, HID))] * (np_ + 1),
        out_specs=[_row_spec(HID)] + [_row_spec(128)] * 2,
        out_shape=[jax.ShapeDtypeStruct((N, HID), _F32)]
                  + [jax.ShapeDtypeStruct((N, 128), _F32)] * 2,
    )(*shs, deg, hb, eW[:128], eW[128:], eb, eg, ebt, *parts,
      *(lax.slice(hWn, (j * HID, 0), ((j + 1) * HID, HID))
        for j in range(np_ + 1)))


def _post_head(shs, deg, hb, eW, eb, eg, ebt, parts,
               W3s, b3, g3, bt3, fcW_pad, fcb_pad):
    np_ = len(parts)

    def body(*refs):
        i = 0
        s_refs = refs[i:i + 2]; i += 2
        deg_ref, hb_ref = refs[i:i + 2]; i += 2
        ew_refs = refs[i:i + 2]; i += 2
        eb_ref, g_ref, bt_ref = refs[i:i + 3]; i += 3
        part_refs = refs[i:i + np_]; i += np_
        w3_refs = refs[i:i + np_ + 1]; i += np_ + 1
        b3_ref, g3_ref, bt3_ref, fw_ref, fb_ref = refs[i:i + 5]; i += 5
        o_ref = refs[i]

        h = _post_val([r[...] for r in s_refs], deg_ref, hb_ref, ew_refs,
                      eb_ref, g_ref, bt_ref)
        t = jnp.dot(h, w3_refs[np_][...], preferred_element_type=_F32)
        for j in range(np_):
            t += jnp.dot(part_refs[j][...], w3_refs[j][...],
                         preferred_element_type=_F32)
        z = _leaky_val(_ln_val(t + b3_ref[...], g3_ref[...], bt3_ref[...]))
        o_ref[...] = jnp.dot(z, fw_ref[...], preferred_element_type=_F32) \
            + fb_ref[...]

    return pl.pallas_call(
        body,
        grid=(GRID,),
        in_specs=_POST_SPECS
                 + [_row_spec(HID)] * np_ + [_full_spec((HID, HID))] * (np_ + 1)
                 + [_full_spec((1, HID))] * 3
                 + [_full_spec((HID, 128)), _full_spec((1, 128))],
        out_specs=_row_spec(128),
        out_shape=jax.ShapeDtypeStruct((N, 128), _F32),
    )(*shs, deg, hb, eW[:128], eW[128:], eb, eg, ebt, *parts, *W3s,
      b3, g3, bt3, fcW_pad, fcb_pad)


def kernel(x_all, edge_index, p, fc1_W, fc1_b, ln1_g, ln1_b,
           hW0, hb0, eW0, eb0, eg0, ebt0,
           hW1, hb1, eW1, eb1, eg1, ebt1,
           hW2, hb2, eW2, eb2, eg2, ebt2,
           fc3_W, fc3_b, ln3_g, ln3_b, fc_W, fc_b):
    del p
    hWs = [hW0, hW1, hW2]
    hbs = [hb0, hb1, hb2]
    eWs = [eW0, eW1, eW2]
    ebs = [eb0, eb1, eb2]
    egs = [eg0, eg1, eg2]
    ebts = [ebt0, ebt1, ebt2]

    row = lambda v: v.reshape(1, -1)

    npad = EP - E
    ar = jnp.arange(npad, dtype=jnp.int32)
    src = jnp.concatenate([edge_index[0], (ar * 37) % N]).reshape(NS, NCHUNK, CHUNK)
    dst = jnp.concatenate([edge_index[1], N + (ar % 16)]).reshape(NS, NCHUNK, CHUNK)

    deg = _deg(dst)
    x0, *yhs = _mm_ln_split(x_all, fc1_W, row(fc1_b), row(ln1_g), row(ln1_b),
                            hW0)

    parts = [x0]
    for i in range(LAYERS - 1):
        shs = _seg_sum(*yhs, src, dst)
        h, *yhs = _post_mm(shs, deg, row(hbs[i]), eWs[i], row(ebs[i]),
                           row(egs[i]), row(ebts[i]), parts, hWs[i + 1])
        parts.append(h)

    shs = _seg_sum(*yhs, src, dst)
    W3s = [lax.slice(fc3_W, (j * HID, 0), ((j + 1) * HID, HID))
           for j in range(LAYERS + 1)]
    fcW_pad = jnp.pad(fc_W, ((0, 0), (0, 128 - NUM_CLASS)))
    fcb_pad = jnp.pad(fc_b, ((0, 128 - NUM_CLASS),))
    out = _post_head(shs, deg, row(hbs[2]), eWs[2], row(ebs[2]),
                     row(egs[2]), row(ebts[2]), parts,
                     W3s, row(fc3_b), row(ln3_g), row(ln3_b),
                     fcW_pad, row(fcb_pad))
    return out[:, :NUM_CLASS]


# speedup vs baseline: 13.2093x; 1.0715x over previous
import jax
import jax.numpy as jnp
from jax import lax
from jax.experimental import pallas as pl
from jax.experimental.pallas import tpu as pltpu
from jax.experimental.pallas import tpu_sc as plsc

HID = 256
N = 10000
NP = 10240
E = 160000
EP = 163840
LAYERS = 3
NUM_CLASS = 2

NS = 16
CHUNK = 128
NCHUNK = EP // NS // CHUNK
RPT = NP // NS
RCH = RPT // CHUNK
NB = 2
SCH = 16
NSTAGE = NCHUNK // SCH

_F32 = jnp.float32

_SC_PARAMS = pltpu.CompilerParams(use_tc_tiling_on_sc=True)


def _make_seg_sum():
    mesh = plsc.VectorSubcoreMesh(core_axis_name="c", subcore_axis_name="s")

    out_type = [jax.ShapeDtypeStruct((NP, 128), _F32)] * 2
    scratch = [
        pltpu.VMEM((NB, SCH, CHUNK), jnp.int32),
        pltpu.VMEM((NB, SCH, CHUNK), jnp.int32),
        pltpu.VMEM((NB, CHUNK, 128), _F32),
        pltpu.VMEM_SHARED((NP, 128), _F32),
        pltpu.SemaphoreType.DMA,
    ] + [pltpu.SemaphoreType.DMA] * NB
    def body(y_lo, y_hi, src3, dst3, out_lo, out_hi,
             srcq, dstq, rows, acc, isem, g0, g1):
        sems = [g0, g1]
        c = lax.axis_index("c")
        s = lax.axis_index("s")
        base = s * RPT

        zeros16 = jnp.zeros((16,), _F32)

        def _zero_row(r, _):
            def _zero_col(cc, _):
                rows[0, r, pl.ds(cc * 16, 16)] = zeros16
                return 0
            return lax.fori_loop(0, 128 // 16, _zero_col, 0)

        lax.fori_loop(0, CHUNK, _zero_row, 0)
        for j in range(RCH):
            pltpu.sync_copy(rows.at[0], acc.at[pl.ds(base + j * CHUNK, CHUNK)])

        def stage_copy(q, ib):
            r0 = q * SCH
            return [pltpu.async_copy(src3.at[s, pl.ds(r0, SCH)], srcq.at[ib],
                                     isem),
                    pltpu.async_copy(dst3.at[s, pl.ds(r0, SCH)], dstq.at[ib],
                                     isem)]

        for d in stage_copy(0, 0):
            d.wait()
        plsc.subcore_barrier()

        def fire(ib, j, b):
            @pl.when(c == 0)
            def _():
                pltpu.async_copy(y_lo.at[srcq.at[ib, j]], rows.at[b], sems[b])

            @pl.when(c == 1)
            def _():
                pltpu.async_copy(y_hi.at[srcq.at[ib, j]], rows.at[b], sems[b])

        def drain(ib, j, b):
            pltpu.make_async_copy(y_lo.at[srcq.at[ib, j]], rows.at[b],
                                  sems[b]).wait()
            pltpu.sync_copy(rows.at[b], acc.at[dstq.at[ib, j]], add=True)

        for b in range(NB):
            fire(0, b, b)
        for q in range(NSTAGE):
            ib = q % 2
            nxt = stage_copy(q + 1, 1 - ib) if q + 1 < NSTAGE else []

            def steady(t, _):
                for b in range(NB):
                    j = t * NB + b
                    drain(ib, j, b)
                    fire(ib, j + NB, b)
                return 0

            lax.fori_loop(0, (SCH - NB) // NB, steady, 0)
            for d in nxt:
                d.wait()
            for b in range(NB):
                drain(ib, SCH - NB + b, b)
                if nxt:
                    fire(1 - ib, b, b)

        plsc.subcore_barrier()

        for j in range(RCH):
            r0 = base + j * CHUNK
            pltpu.sync_copy(acc.at[pl.ds(r0, CHUNK)], rows.at[0])

            @pl.when(c == 0)
            def _():
                pltpu.sync_copy(rows.at[0], out_lo.at[pl.ds(r0, CHUNK)])

            @pl.when(c == 1)
            def _():
                pltpu.sync_copy(rows.at[0], out_hi.at[pl.ds(r0, CHUNK)])

    return pl.kernel(body, out_type=out_type, mesh=mesh, scratch_types=scratch,
                     compiler_params=_SC_PARAMS,
                     cost_estimate=pl.CostEstimate(
                         flops=2 * EP * 128, transcendentals=0,
                         bytes_accessed=4 * EP * 128 * 4))


def _make_deg():
    mesh = plsc.VectorSubcoreMesh(core_axis_name="c", subcore_axis_name="s")
    scratch = [
        pltpu.VMEM((NCHUNK, CHUNK), jnp.int32),
        pltpu.VMEM((CHUNK, 16), _F32),
        pltpu.VMEM((CHUNK, 16), _F32),
        pltpu.VMEM_SHARED((NP, 16), _F32),
    ]

    def body(dst3, deg_out, dstb, ones_v, zb, deg_acc):
        c = lax.axis_index("c")
        s = lax.axis_index("s")
        base = s * RPT
        zeros16 = jnp.zeros((16,), _F32)
        ones16 = jnp.ones((16,), _F32)

        def _fill(r, _):
            ones_v[r, :] = ones16
            zb[r, :] = zeros16
            return 0

        lax.fori_loop(0, CHUNK, _fill, 0)
        pltpu.sync_copy(dst3.at[s], dstb)
        for j in range(RCH):
            pltpu.sync_copy(zb, deg_acc.at[pl.ds(base + j * CHUNK, CHUNK)])
        plsc.subcore_barrier()

        @pl.when(c == 0)
        def _():
            def step(k, _):
                pltpu.sync_copy(ones_v, deg_acc.at[dstb.at[k]], add=True)
                return 0
            lax.fori_loop(0, NCHUNK, step, 0)

        plsc.subcore_barrier()

        @pl.when(c == 0)
        def _():
            for j in range(RCH):
                r0 = base + j * CHUNK
                pltpu.sync_copy(deg_acc.at[pl.ds(r0, CHUNK)], zb)
                pltpu.sync_copy(zb, deg_out.at[pl.ds(r0, CHUNK)])

    return pl.kernel(body, out_type=jax.ShapeDtypeStruct((NP, 16), _F32),
                     mesh=mesh, scratch_types=scratch,
                     compiler_params=_SC_PARAMS,
                     cost_estimate=pl.CostEstimate(
                         flops=EP * 16, transcendentals=0,
                         bytes_accessed=2 * EP * 16 * 4))


_seg_sum = _make_seg_sum()
_deg = _make_deg()


BR = 1000
GRID = N // BR


def _ln_val(t, g, b):
    mu = jnp.mean(t, axis=-1, keepdims=True)
    d = t - mu
    var = jnp.mean(d * d, axis=-1, keepdims=True)
    return d * lax.rsqrt(var + 1e-5) * g + b


def _leaky_val(t):
    return jnp.where(t >= 0, t, 0.01 * t)


def _row_spec(width):
    return pl.BlockSpec((BR, width), lambda i: (i, 0))


def _full_spec(shape):
    return pl.BlockSpec(shape, lambda i: (0,) * len(shape))


def _halves(y, os):
    os[0][...] = y[:, :128]
    os[1][...] = y[:, 128:]


def _mm_ln_split(x, W, b, g, bt, hW):
    K = x.shape[1]

    def body(x_ref, w_ref, b_ref, g_ref, bt_ref, hw_ref, o_ref, *oh):
        t = jnp.dot(x_ref[...], w_ref[...], preferred_element_type=_F32)
        x0 = _ln_val(t + b_ref[...], g_ref[...], bt_ref[...])
        o_ref[...] = x0
        _halves(jnp.dot(x0, hw_ref[...], preferred_element_type=_F32), oh)

    return pl.pallas_call(
        body,
        grid=(GRID,),
        in_specs=[_row_spec(K), _full_spec((K, HID)), _full_spec((1, HID)),
                  _full_spec((1, HID)), _full_spec((1, HID)),
                  _full_spec((HID, HID))],
        out_specs=[_row_spec(HID)] + [_row_spec(128)] * 2,
        out_shape=[jax.ShapeDtypeStruct((N, HID), _F32)]
                  + [jax.ShapeDtypeStruct((N, 128), _F32)] * 2,
    )(x, W, b, g, bt, hW)


def _post_val(s_vals, deg_ref, hb_ref, w_refs, eb_ref, g_ref, bt_ref):
    hb = hb_ref[...]
    t = jnp.dot(s_vals[0], w_refs[0][...], preferred_element_type=_F32)
    bias = jnp.dot(hb[:, :128], w_refs[0][...], preferred_element_type=_F32)
    t += jnp.dot(s_vals[1], w_refs[1][...], preferred_element_type=_F32)
    bias += jnp.dot(hb[:, 128:], w_refs[1][...], preferred_element_type=_F32)
    inv = 1.0 / jnp.maximum(deg_ref[:, 0:1], 1.0)
    t = t * inv + bias + eb_ref[...]
    return _leaky_val(_ln_val(t, g_ref[...], bt_ref[...]))


_POST_SPECS = ([_row_spec(128)] * 2 + [_row_spec(16), _full_spec((1, HID))]
               + [_full_spec((128, HID))] * 2 + [_full_spec((1, HID))] * 3)


def _mm_parts(parts, Ws):
    np_ = len(parts)

    def body(*refs):
        part_refs, w_refs, o_ref = refs[:np_], refs[np_:2 * np_], refs[-1]
        t = jnp.dot(part_refs[0][...], w_refs[0][...],
                    preferred_element_type=_F32)
        for j in range(1, np_):
            t += jnp.dot(part_refs[j][...], w_refs[j][...],
                         preferred_element_type=_F32)
        o_ref[...] = t

    return pl.pallas_call(
        body,
        grid=(GRID,),
        in_specs=[_row_spec(HID)] * np_ + [_full_spec((HID, HID))] * np_,
        out_specs=_row_spec(HID),
        out_shape=jax.ShapeDtypeStruct((N, HID), _F32),
    )(*parts, *Ws)


def _post_mm(shs, deg, hb, eW, eb, eg, ebt, ypart, hW_last):
    def body(*refs):
        i = 0
        s_refs = refs[i:i + 2]; i += 2
        deg_ref, hb_ref = refs[i:i + 2]; i += 2
        ew_refs = refs[i:i + 2]; i += 2
        eb_ref, g_ref, bt_ref = refs[i:i + 3]; i += 3
        yp_ref, hw_ref = refs[i:i + 2]; i += 2
        h_ref = refs[i]; i += 1
        oh = refs[i:]

        h = _post_val([r[...] for r in s_refs], deg_ref, hb_ref, ew_refs,
                      eb_ref, g_ref, bt_ref)
        h_ref[...] = h
        y = yp_ref[...] + jnp.dot(h, hw_ref[...], preferred_element_type=_F32)
        _halves(y, oh)

    return pl.pallas_call(
        body,
        grid=(GRID,),
        in_specs=_POST_SPECS + [_row_spec(HID), _full_spec((HID, HID))],
        out_specs=[_row_spec(HID)] + [_row_spec(128)] * 2,
        out_shape=[jax.ShapeDtypeStruct((N, HID), _F32)]
                  + [jax.ShapeDtypeStruct((N, 128), _F32)] * 2,
    )(*shs, deg, hb, eW[:128], eW[128:], eb, eg, ebt, ypart, hW_last)


def _post_head(shs, deg, hb, eW, eb, eg, ebt, tpart, W3_last,
               b3, g3, bt3, fcW_pad, fcb_pad):
    def body(*refs):
        i = 0
        s_refs = refs[i:i + 2]; i += 2
        deg_ref, hb_ref = refs[i:i + 2]; i += 2
        ew_refs = refs[i:i + 2]; i += 2
        eb_ref, g_ref, bt_ref = refs[i:i + 3]; i += 3
        (tp_ref, w3_ref, b3_ref, g3_ref, bt3_ref,
         fw_ref, fb_ref, o_ref) = refs[i:]

        h = _post_val([r[...] for r in s_refs], deg_ref, hb_ref, ew_refs,
                      eb_ref, g_ref, bt_ref)
        t = tp_ref[...] + jnp.dot(h, w3_ref[...], preferred_element_type=_F32)
        z = _leaky_val(_ln_val(t + b3_ref[...], g3_ref[...], bt3_ref[...]))
        o_ref[...] = jnp.dot(z, fw_ref[...], preferred_element_type=_F32) \
            + fb_ref[...]

    return pl.pallas_call(
        body,
        grid=(GRID,),
        in_specs=_POST_SPECS
                 + [_row_spec(HID), _full_spec((HID, HID))]
                 + [_full_spec((1, HID))] * 3
                 + [_full_spec((HID, 128)), _full_spec((1, 128))],
        out_specs=_row_spec(128),
        out_shape=jax.ShapeDtypeStruct((N, 128), _F32),
    )(*shs, deg, hb, eW[:128], eW[128:], eb, eg, ebt, tpart, W3_last,
      b3, g3, bt3, fcW_pad, fcb_pad)


def kernel(x_all, edge_index, p, fc1_W, fc1_b, ln1_g, ln1_b,
           hW0, hb0, eW0, eb0, eg0, ebt0,
           hW1, hb1, eW1, eb1, eg1, ebt1,
           hW2, hb2, eW2, eb2, eg2, ebt2,
           fc3_W, fc3_b, ln3_g, ln3_b, fc_W, fc_b):
    del p
    hWs = [hW0, hW1, hW2]
    hbs = [hb0, hb1, hb2]
    eWs = [eW0, eW1, eW2]
    ebs = [eb0, eb1, eb2]
    egs = [eg0, eg1, eg2]
    ebts = [ebt0, ebt1, ebt2]

    row = lambda v: v.reshape(1, -1)

    npad = EP - E
    ar = jnp.arange(npad, dtype=jnp.int32)
    src = jnp.concatenate([edge_index[0], (ar * 37) % N]).reshape(NS, NCHUNK, CHUNK)
    dst = jnp.concatenate([edge_index[1], N + (ar % 16)]).reshape(NS, NCHUNK, CHUNK)

    deg = _deg(dst)
    x0, *yhs = _mm_ln_split(x_all, fc1_W, row(fc1_b), row(ln1_g), row(ln1_b),
                            hW0)

    def wsplit(W, n):
        return [lax.slice(W, (j * HID, 0), ((j + 1) * HID, HID))
                for j in range(n)]

    parts = [x0]
    for i in range(LAYERS - 1):
        shs = _seg_sum(*yhs, src, dst)
        hWn = wsplit(hWs[i + 1], i + 2)
        ypart = _mm_parts(parts, hWn[:i + 1])
        h, *yhs = _post_mm(shs, deg, row(hbs[i]), eWs[i], row(ebs[i]),
                           row(egs[i]), row(ebts[i]), ypart, hWn[i + 1])
        parts.append(h)

    shs = _seg_sum(*yhs, src, dst)
    W3s = wsplit(fc3_W, LAYERS + 1)
    tpart = _mm_parts(parts, W3s[:LAYERS])
    fcW_pad = jnp.pad(fc_W, ((0, 0), (0, 128 - NUM_CLASS)))
    fcb_pad = jnp.pad(fc_b, ((0, 128 - NUM_CLASS),))
    out = _post_head(shs, deg, row(hbs[2]), eWs[2], row(ebs[2]),
                     row(egs[2]), row(ebts[2]), tpart, W3s[LAYERS],
                     row(fc3_b), row(ln3_g), row(ln3_b),
                     fcW_pad, row(fcb_pad))
    return out[:, :NUM_CLASS]
